# baseline (speedup 1.0000x reference)
import functools

import jax
import jax.numpy as jnp
import numpy as np
from jax import lax
from jax.experimental import pallas as pl
from jax.experimental.pallas import tpu as pltpu
from jax.experimental.pallas import tpu_sc as plsc

_DIM_LEN = 768
_GROUP = 4

_B = 2
_R = _B * _DIM_LEN
_D = 224 * 224
_NC = 2
_NS = 16
_NW = _NC * _NS
_RPW = _R // _NW
_CHUNK = 2
_STEPS = _RPW // _CHUNK


def _src_rows() -> np.ndarray:
    with jax.ensure_compile_time_eval():
        perm = np.asarray(jax.random.permutation(jax.random.key(1), _DIM_LEN // _GROUP))
    chan = (perm[:, None] * _GROUP + np.arange(_GROUP)[None, :]).reshape(-1)
    rows = (np.arange(_B)[:, None] * _DIM_LEN + chan[None, :]).reshape(-1)
    return rows.astype(np.int32)


_SRC_ROWS = _src_rows()


def _permute_rows(x2, src):
    mesh = plsc.VectorSubcoreMesh(core_axis_name="c", subcore_axis_name="s")

    @functools.partial(
        pl.kernel,
        mesh=mesh,
        out_type=jax.ShapeDtypeStruct((_R, _D), jnp.float32),
        scratch_types=[
            pltpu.VMEM((_STEPS, _CHUNK), jnp.int32),
            pltpu.VMEM((_CHUNK, _D), jnp.float32),
            pltpu.SemaphoreType.DMA,
            pltpu.SemaphoreType.DMA,
        ],
    )
    def k(x_hbm, src_hbm, out_hbm, idx_v, buf, gsem, wsem):
        wid = lax.axis_index("s") * _NC + lax.axis_index("c")
        base = wid * _RPW
        pltpu.sync_copy(src_hbm.at[pl.ds(wid * _STEPS, _STEPS)], idx_v)

        def step(t, carry):
            g = pltpu.async_copy(x_hbm.at[idx_v.at[t]], buf, gsem)
            g.wait()
            w = pltpu.async_copy(buf, out_hbm.at[pl.ds(base + t * _CHUNK, _CHUNK)], wsem)
            w.wait()
            return carry

        lax.fori_loop(0, _STEPS, step, 0)

    return k(x2, src)


def kernel(x):
    x2 = x.reshape(_R, _D)
    src = jnp.asarray(_SRC_ROWS).reshape(_R // _CHUNK, _CHUNK)
    out2 = _permute_rows(x2, src)
    return out2.reshape(x.shape)


# speedup vs baseline: 1.0055x; 1.0055x over previous
import functools

import jax
import jax.numpy as jnp
import numpy as np
from jax import lax
from jax.experimental import pallas as pl
from jax.experimental.pallas import tpu as pltpu
from jax.experimental.pallas import tpu_sc as plsc

_DIM_LEN = 768
_GROUP = 4

_B = 2
_R = _B * _DIM_LEN
_D = 224 * 224
_NC = 2
_NS = 16
_NW = _NC * _NS
_RPW = _R // _NW
_NBUF = 2
_STEPS = _RPW // _NBUF


def _src_rows() -> np.ndarray:
    with jax.ensure_compile_time_eval():
        perm = np.asarray(jax.random.permutation(jax.random.key(1), _DIM_LEN // _GROUP))
    chan = (perm[:, None] * _GROUP + np.arange(_GROUP)[None, :]).reshape(-1)
    rows = (np.arange(_B)[:, None] * _DIM_LEN + chan[None, :]).reshape(-1)
    return rows.astype(np.int32)


_SRC_ROWS = _src_rows()


def _permute_rows(x2, src):
    mesh = plsc.VectorSubcoreMesh(core_axis_name="c", subcore_axis_name="s")

    @functools.partial(
        pl.kernel,
        mesh=mesh,
        out_type=jax.ShapeDtypeStruct((_R, _D), jnp.float32),
        scratch_types=[
            pltpu.VMEM((_RPW, 1), jnp.int32),
            pltpu.VMEM((1, _D), jnp.float32),
            pltpu.VMEM((1, _D), jnp.float32),
            pltpu.SemaphoreType.DMA,
            pltpu.SemaphoreType.DMA,
            pltpu.SemaphoreType.DMA,
            pltpu.SemaphoreType.DMA,
        ],
    )
    def k(x_hbm, src_hbm, out_hbm, idx_v, buf0, buf1, gsem0, gsem1, wsem0, wsem1):
        wid = lax.axis_index("s") * _NC + lax.axis_index("c")
        base = wid * _RPW
        bufs = (buf0, buf1)
        gsems = (gsem0, gsem1)
        wsems = (wsem0, wsem1)
        pltpu.sync_copy(src_hbm.at[pl.ds(base, _RPW)], idx_v)

        for b in range(_NBUF):
            pltpu.async_copy(x_hbm.at[idx_v.at[b]], bufs[b], gsems[b])

        def step(t, carry):
            for b in range(_NBUF):
                i = t * _NBUF + b
                pltpu.make_async_copy(x_hbm.at[idx_v.at[i]], bufs[b], gsems[b]).wait()
                w = pltpu.async_copy(bufs[b], out_hbm.at[pl.ds(base + i, 1)], wsems[b])
                w.wait()
                nxt = i + _NBUF

                @pl.when(nxt < _RPW)
                def _():
                    pltpu.async_copy(x_hbm.at[idx_v.at[nxt]], bufs[b], gsems[b])

            return carry

        lax.fori_loop(0, _STEPS, step, 0)

    return k(x2, src)


def kernel(x):
    x2 = x.reshape(_R, _D)
    src = jnp.asarray(_SRC_ROWS).reshape(_R, 1)
    out2 = _permute_rows(x2, src)
    return out2.reshape(x.shape)
